# baseline (speedup 1.0000x reference)
import functools

import jax
import jax.numpy as jnp
from jax import lax
from jax.experimental import pallas as pl
from jax.experimental.pallas import tpu as pltpu
from jax.experimental.pallas import tpu_sc as plsc

E = 8
TOP_K = 2
D = 1024
I = 512
S = 2048
T = 2048

BM = 256
MT = 24
M_PAD = MT * BM
CB = 256
NC, NS = 2, 16
NW = NC * NS
ROWS_W = T // NW


def _router_kernel(x_ref, rw_ref, pos1_ref, pos2_ref, w1_ref, w2_ref, gid_ref):
    xb = x_ref[...]
    logits = jax.lax.dot_general(
        xb.astype(jnp.bfloat16), rw_ref[...].astype(jnp.bfloat16),
        (((1,), (1,)), ((), ())),
        preferred_element_type=jnp.float32)
    m = jnp.max(logits, axis=-1, keepdims=True)
    ex = jnp.exp(logits - m)
    probs = ex / jnp.sum(ex, axis=-1, keepdims=True)

    eids = jax.lax.broadcasted_iota(jnp.int32, (T, E), 1)
    i1 = jnp.argmax(probs, axis=-1)[:, None]
    oh1 = eids == i1
    v1 = jnp.max(probs, axis=-1, keepdims=True)
    masked = jnp.where(oh1, -jnp.inf, probs)
    i2 = jnp.argmax(masked, axis=-1)[:, None]
    oh2 = eids == i2
    v2 = jnp.max(masked, axis=-1, keepdims=True)
    denom = v1 + v2
    w1_ref[...] = v1 / denom
    w2_ref[...] = v2 / denom

    a = (oh1 | oh2).astype(jnp.float32)
    row = jax.lax.broadcasted_iota(jnp.int32, (CB, CB), 0)
    col = jax.lax.broadcasted_iota(jnp.int32, (CB, CB), 1)
    tri = (row >= col).astype(jnp.bfloat16)
    running = jnp.zeros((1, E), dtype=jnp.float32)
    cex_blocks = []
    for b in range(T // CB):
        ab = a[b * CB:(b + 1) * CB]
        cin = jax.lax.dot_general(
            tri, ab.astype(jnp.bfloat16), (((1,), (0,)), ((), ())),
            preferred_element_type=jnp.float32)
        cex_blocks.append(cin - ab + running)
        running = running + jnp.sum(ab, axis=0, keepdims=True)
    cex = jnp.concatenate(cex_blocks, axis=0)
    counts = running

    nt = jnp.floor((counts + (BM - 1)) * (1.0 / BM))
    erow = jax.lax.broadcasted_iota(jnp.int32, (E, E), 0)
    ecol = jax.lax.broadcasted_iota(jnp.int32, (E, E), 1)
    strict = (erow < ecol).astype(jnp.bfloat16)
    tstart = jax.lax.dot_general(
        nt.astype(jnp.bfloat16), strict, (((1,), (0,)), ((), ())),
        preferred_element_type=jnp.float32)
    padded_off = tstart * float(BM)

    pmat = cex + padded_off
    pos1 = jnp.sum(pmat * oh1.astype(jnp.float32), axis=1, keepdims=True)
    pos2 = jnp.sum(pmat * oh2.astype(jnp.float32), axis=1, keepdims=True)
    pos1_ref[...] = pos1.astype(jnp.int32)
    pos2_ref[...] = pos2.astype(jnp.int32)

    iota_mt = jax.lax.broadcasted_iota(jnp.int32, (1, MT), 1).astype(jnp.float32)
    gidf = jnp.zeros((1, MT), dtype=jnp.float32)
    for e in range(E):
        gidf = gidf + (iota_mt >= tstart[:, e:e + 1]).astype(jnp.float32)
    gid_ref[...] = (gidf - 1.0).astype(jnp.int32)


def _router_call(x, router_weight):
    return pl.pallas_call(
        _router_kernel,
        grid=(1,),
        in_specs=[
            pl.BlockSpec((T, D), lambda i: (0, 0)),
            pl.BlockSpec((E, D), lambda i: (0, 0)),
        ],
        out_specs=[
            pl.BlockSpec((T, 1), lambda i: (0, 0)),
            pl.BlockSpec((T, 1), lambda i: (0, 0)),
            pl.BlockSpec((T, 1), lambda i: (0, 0)),
            pl.BlockSpec((T, 1), lambda i: (0, 0)),
            pl.BlockSpec((1, MT), lambda i: (0, 0)),
        ],
        out_shape=[
            jax.ShapeDtypeStruct((T, 1), jnp.int32),
            jax.ShapeDtypeStruct((T, 1), jnp.int32),
            jax.ShapeDtypeStruct((T, 1), jnp.float32),
            jax.ShapeDtypeStruct((T, 1), jnp.float32),
            jax.ShapeDtypeStruct((1, MT), jnp.int32),
        ],
    )(x, router_weight)


def _sc_dispatch(x, pos1, pos2):
    mesh = plsc.VectorSubcoreMesh(core_axis_name="c", subcore_axis_name="s")

    @functools.partial(
        pl.kernel, mesh=mesh,
        out_type=jax.ShapeDtypeStruct((M_PAD, D), jnp.float32),
        scratch_types=[
            pltpu.VMEM((ROWS_W,), jnp.int32),
            pltpu.VMEM((ROWS_W,), jnp.int32),
            pltpu.VMEM((ROWS_W, D), jnp.float32),
            pltpu.SemaphoreType.DMA,
            pltpu.SemaphoreType.DMA,
        ],
    )
    def k(x_hbm, i1_hbm, i2_hbm, out_hbm, idx1_v, idx2_v, rows_v, sem1, sem2):
        wid = lax.axis_index("s") * NC + lax.axis_index("c")
        base = wid * ROWS_W
        pltpu.sync_copy(i1_hbm.at[pl.ds(base, ROWS_W)], idx1_v)
        pltpu.sync_copy(i2_hbm.at[pl.ds(base, ROWS_W)], idx2_v)
        pltpu.sync_copy(x_hbm.at[pl.ds(base, ROWS_W)], rows_v)
        c1 = pltpu.async_copy(rows_v, out_hbm.at[idx1_v], sem1)
        c2 = pltpu.async_copy(rows_v, out_hbm.at[idx2_v], sem2)
        c1.wait()
        c2.wait()

    return k(x, pos1, pos2)


def _expert_kernel(gid_ref, xs_ref, egu_ref, ed_ref, eo_ref):
    xs_raw = xs_ref[...]
    xs_fin = jnp.where(jnp.abs(xs_raw) < 1e30, xs_raw, 0.0)
    xs = xs_fin.astype(jnp.bfloat16)
    gu = jax.lax.dot_general(
        xs, egu_ref[0], (((1,), (1,)), ((), ())),
        preferred_element_type=jnp.float32)
    h = jax.nn.silu(gu[:, :I]) * gu[:, I:]
    eo = jax.lax.dot_general(
        h.astype(jnp.bfloat16), ed_ref[0], (((1,), (1,)), ((), ())),
        preferred_element_type=jnp.float32)
    eo_ref[...] = eo.astype(jnp.bfloat16)


def _expert_call(gid, xs, egu_bf, ed_bf):
    grid_spec = pltpu.PrefetchScalarGridSpec(
        num_scalar_prefetch=1,
        grid=(MT,),
        in_specs=[
            pl.BlockSpec((BM, D), lambda i, g: (i, 0)),
            pl.BlockSpec((1, 2 * I, D), lambda i, g: (g[i], 0, 0)),
            pl.BlockSpec((1, D, I), lambda i, g: (g[i], 0, 0)),
        ],
        out_specs=pl.BlockSpec((BM, D), lambda i, g: (i, 0)),
    )
    return pl.pallas_call(
        _expert_kernel,
        grid_spec=grid_spec,
        out_shape=jax.ShapeDtypeStruct((M_PAD, D), jnp.bfloat16),
    )(gid, xs, egu_bf, ed_bf)


BT = 256


def _shared_kernel(x_ref, sg_ref, su_ref, sd_ref, seg_ref, eo_ref,
                   p1_ref, p2_ref, w1_ref, w2_ref, out_ref):
    xb = x_ref[...]
    xb_bf = xb.astype(jnp.bfloat16)
    sg = jax.lax.dot_general(
        xb_bf, sg_ref[...], (((1,), (0,)), ((), ())),
        preferred_element_type=jnp.float32)
    su = jax.lax.dot_general(
        xb_bf, su_ref[...], (((1,), (0,)), ((), ())),
        preferred_element_type=jnp.float32)
    sh = jax.nn.silu(sg) * su
    so = jax.lax.dot_general(
        sh.astype(jnp.bfloat16), sd_ref[...], (((1,), (0,)), ((), ())),
        preferred_element_type=jnp.float32)
    glogit = jax.lax.dot_general(
        xb, seg_ref[...], (((1,), (1,)), ((), ())),
        preferred_element_type=jnp.float32)
    slots = jax.lax.broadcasted_iota(jnp.int32, (BT, M_PAD), 1).astype(jnp.float32)
    p1f = p1_ref[...].astype(jnp.float32)
    p2f = p2_ref[...].astype(jnp.float32)
    oh32 = (jnp.where(slots == p1f, w1_ref[...], 0.0)
            + jnp.where(slots == p2f, w2_ref[...], 0.0))
    oh = oh32.astype(jnp.bfloat16)
    moe = jax.lax.dot_general(
        oh, eo_ref[...], (((1,), (0,)), ((), ())),
        preferred_element_type=jnp.float32)
    out_ref[...] = jax.nn.sigmoid(glogit) * so + moe


def _shared_call(x, sg_bf, su_bf, sd_bf, seg, eo, pos1, pos2, w1, w2):
    return pl.pallas_call(
        _shared_kernel,
        grid=(T // BT,),
        in_specs=[
            pl.BlockSpec((BT, D), lambda i: (i, 0)),
            pl.BlockSpec((D, S), lambda i: (0, 0)),
            pl.BlockSpec((D, S), lambda i: (0, 0)),
            pl.BlockSpec((S, D), lambda i: (0, 0)),
            pl.BlockSpec((1, D), lambda i: (0, 0)),
            pl.BlockSpec((M_PAD, D), lambda i: (0, 0)),
            pl.BlockSpec((BT, 1), lambda i: (i, 0)),
            pl.BlockSpec((BT, 1), lambda i: (i, 0)),
            pl.BlockSpec((BT, 1), lambda i: (i, 0)),
            pl.BlockSpec((BT, 1), lambda i: (i, 0)),
        ],
        out_specs=pl.BlockSpec((BT, D), lambda i: (i, 0)),
        out_shape=jax.ShapeDtypeStruct((T, D), jnp.float32),
    )(x, sg_bf, su_bf, sd_bf, seg, eo, pos1, pos2, w1, w2)


def kernel(x, router_weight, expert_gate_up, expert_down, shared_gate,
           shared_up, shared_down, shared_expert_gate):
    egu_bf = expert_gate_up.astype(jnp.bfloat16)
    ed_bf = expert_down.astype(jnp.bfloat16)
    sg_bf = shared_gate.astype(jnp.bfloat16)
    su_bf = shared_up.astype(jnp.bfloat16)
    sd_bf = shared_down.astype(jnp.bfloat16)
    seg = shared_expert_gate.reshape(1, D)

    pos1, pos2, w1, w2, gid = _router_call(x, router_weight)
    pos1_f = pos1.reshape(T)
    pos2_f = pos2.reshape(T)
    gid_f = gid.reshape(MT)

    xs = _sc_dispatch(x, pos1_f, pos2_f)
    eo = _expert_call(gid_f, xs, egu_bf, ed_bf)
    return _shared_call(x, sg_bf, su_bf, sd_bf, seg, eo, pos1, pos2, w1, w2)


# speedup vs baseline: 1.0840x; 1.0840x over previous
import functools

import jax
import jax.numpy as jnp
from jax import lax
from jax.experimental import pallas as pl
from jax.experimental.pallas import tpu as pltpu
from jax.experimental.pallas import tpu_sc as plsc

E = 8
TOP_K = 2
D = 1024
I = 512
S = 2048
T = 2048

BM = 256
MT = 24
M_PAD = MT * BM
CB = 256
NC, NS = 2, 16
NW = NC * NS
ROWS_W = T // NW


def _router_kernel(x_ref, rw_ref, pos1_ref, pos2_ref, w1_ref, w2_ref, gid_ref):
    xb = x_ref[...]
    logits = jax.lax.dot_general(
        xb.astype(jnp.bfloat16), rw_ref[...].astype(jnp.bfloat16),
        (((1,), (1,)), ((), ())),
        preferred_element_type=jnp.float32)
    m = jnp.max(logits, axis=-1, keepdims=True)
    ex = jnp.exp(logits - m)
    probs = ex / jnp.sum(ex, axis=-1, keepdims=True)

    eids = jax.lax.broadcasted_iota(jnp.int32, (T, E), 1)
    i1 = jnp.argmax(probs, axis=-1)[:, None]
    oh1 = eids == i1
    v1 = jnp.max(probs, axis=-1, keepdims=True)
    masked = jnp.where(oh1, -jnp.inf, probs)
    i2 = jnp.argmax(masked, axis=-1)[:, None]
    oh2 = eids == i2
    v2 = jnp.max(masked, axis=-1, keepdims=True)
    denom = v1 + v2
    w1_ref[...] = v1 / denom
    w2_ref[...] = v2 / denom

    a = (oh1 | oh2).astype(jnp.float32)
    row = jax.lax.broadcasted_iota(jnp.int32, (CB, CB), 0)
    col = jax.lax.broadcasted_iota(jnp.int32, (CB, CB), 1)
    tri = (row >= col).astype(jnp.bfloat16)
    running = jnp.zeros((1, E), dtype=jnp.float32)
    cex_blocks = []
    for b in range(T // CB):
        ab = a[b * CB:(b + 1) * CB]
        cin = jax.lax.dot_general(
            tri, ab.astype(jnp.bfloat16), (((1,), (0,)), ((), ())),
            preferred_element_type=jnp.float32)
        cex_blocks.append(cin - ab + running)
        running = running + jnp.sum(ab, axis=0, keepdims=True)
    cex = jnp.concatenate(cex_blocks, axis=0)
    counts = running

    nt = jnp.floor((counts + (BM - 1)) * (1.0 / BM))
    erow = jax.lax.broadcasted_iota(jnp.int32, (E, E), 0)
    ecol = jax.lax.broadcasted_iota(jnp.int32, (E, E), 1)
    strict = (erow < ecol).astype(jnp.bfloat16)
    tstart = jax.lax.dot_general(
        nt.astype(jnp.bfloat16), strict, (((1,), (0,)), ((), ())),
        preferred_element_type=jnp.float32)
    padded_off = tstart * float(BM)

    pmat = cex + padded_off
    pos1 = jnp.sum(pmat * oh1.astype(jnp.float32), axis=1, keepdims=True)
    pos2 = jnp.sum(pmat * oh2.astype(jnp.float32), axis=1, keepdims=True)
    pos1_ref[...] = pos1.astype(jnp.int32)
    pos2_ref[...] = pos2.astype(jnp.int32)

    iota_mt = jax.lax.broadcasted_iota(jnp.int32, (1, MT), 1).astype(jnp.float32)
    gidf = jnp.zeros((1, MT), dtype=jnp.float32)
    for e in range(E):
        gidf = gidf + (iota_mt >= tstart[:, e:e + 1]).astype(jnp.float32)
    gid_ref[...] = (gidf - 1.0).astype(jnp.int32)


def _router_call(x, router_weight):
    return pl.pallas_call(
        _router_kernel,
        grid=(1,),
        in_specs=[
            pl.BlockSpec((T, D), lambda i: (0, 0)),
            pl.BlockSpec((E, D), lambda i: (0, 0)),
        ],
        out_specs=[
            pl.BlockSpec((T, 1), lambda i: (0, 0)),
            pl.BlockSpec((T, 1), lambda i: (0, 0)),
            pl.BlockSpec((T, 1), lambda i: (0, 0)),
            pl.BlockSpec((T, 1), lambda i: (0, 0)),
            pl.BlockSpec((1, MT), lambda i: (0, 0)),
        ],
        out_shape=[
            jax.ShapeDtypeStruct((T, 1), jnp.int32),
            jax.ShapeDtypeStruct((T, 1), jnp.int32),
            jax.ShapeDtypeStruct((T, 1), jnp.float32),
            jax.ShapeDtypeStruct((T, 1), jnp.float32),
            jax.ShapeDtypeStruct((1, MT), jnp.int32),
        ],
    )(x, router_weight)


def _sc_dispatch(x, pos1, pos2):
    mesh = plsc.VectorSubcoreMesh(core_axis_name="c", subcore_axis_name="s")

    @functools.partial(
        pl.kernel, mesh=mesh,
        out_type=jax.ShapeDtypeStruct((M_PAD, D), jnp.float32),
        scratch_types=[
            pltpu.VMEM((ROWS_W,), jnp.int32),
            pltpu.VMEM((ROWS_W,), jnp.int32),
            pltpu.VMEM((ROWS_W, D), jnp.float32),
            pltpu.SemaphoreType.DMA,
            pltpu.SemaphoreType.DMA,
        ],
    )
    def k(x_hbm, i1_hbm, i2_hbm, out_hbm, idx1_v, idx2_v, rows_v, sem1, sem2):
        wid = lax.axis_index("s") * NC + lax.axis_index("c")
        base = wid * ROWS_W
        pltpu.sync_copy(i1_hbm.at[pl.ds(base, ROWS_W)], idx1_v)
        pltpu.sync_copy(i2_hbm.at[pl.ds(base, ROWS_W)], idx2_v)
        pltpu.sync_copy(x_hbm.at[pl.ds(base, ROWS_W)], rows_v)
        c1 = pltpu.async_copy(rows_v, out_hbm.at[idx1_v], sem1)
        c2 = pltpu.async_copy(rows_v, out_hbm.at[idx2_v], sem2)
        c1.wait()
        c2.wait()

    return k(x, pos1, pos2)


def _expert_kernel(gid_ref, xs_ref, egu_ref, ed_ref, eo_ref):
    xs = xs_ref[...].astype(jnp.bfloat16)
    gu = jax.lax.dot_general(
        xs, egu_ref[0], (((1,), (1,)), ((), ())),
        preferred_element_type=jnp.float32)
    h = jax.nn.silu(gu[:, :I]) * gu[:, I:]
    eo = jax.lax.dot_general(
        h.astype(jnp.bfloat16), ed_ref[0], (((1,), (1,)), ((), ())),
        preferred_element_type=jnp.float32)
    eo_ref[...] = eo


def _expert_call(gid, xs, egu_bf, ed_bf):
    grid_spec = pltpu.PrefetchScalarGridSpec(
        num_scalar_prefetch=1,
        grid=(MT,),
        in_specs=[
            pl.BlockSpec((BM, D), lambda i, g: (i, 0)),
            pl.BlockSpec((1, 2 * I, D), lambda i, g: (g[i], 0, 0)),
            pl.BlockSpec((1, D, I), lambda i, g: (g[i], 0, 0)),
        ],
        out_specs=pl.BlockSpec((BM, D), lambda i, g: (i, 0)),
    )
    return pl.pallas_call(
        _expert_kernel,
        grid_spec=grid_spec,
        out_shape=jax.ShapeDtypeStruct((M_PAD, D), jnp.float32),
    )(gid, xs, egu_bf, ed_bf)


def _sc_combine(eo, pos1, pos2):
    mesh = plsc.VectorSubcoreMesh(core_axis_name="c", subcore_axis_name="s")
    row_t = jax.ShapeDtypeStruct((T, D), jnp.float32)

    @functools.partial(
        pl.kernel, mesh=mesh,
        out_type=(row_t, row_t),
        scratch_types=[
            pltpu.VMEM((ROWS_W,), jnp.int32),
            pltpu.VMEM((ROWS_W,), jnp.int32),
            pltpu.VMEM((ROWS_W // 2, D), jnp.float32),
            pltpu.VMEM((ROWS_W // 2, D), jnp.float32),
            pltpu.SemaphoreType.DMA,
            pltpu.SemaphoreType.DMA,
        ],
    )
    def k(eo_hbm, i1_hbm, i2_hbm, g0_hbm, g1_hbm, idx1_v, idx2_v,
          r0_v, r1_v, sem1, sem2):
        wid = lax.axis_index("s") * NC + lax.axis_index("c")
        base = wid * ROWS_W
        half = ROWS_W // 2
        pltpu.sync_copy(i1_hbm.at[pl.ds(base, ROWS_W)], idx1_v)
        pltpu.sync_copy(i2_hbm.at[pl.ds(base, ROWS_W)], idx2_v)
        for chunk in range(2):
            lo = chunk * half
            c1 = pltpu.async_copy(
                eo_hbm.at[idx1_v.at[pl.ds(lo, half)]], r0_v, sem1)
            c2 = pltpu.async_copy(
                eo_hbm.at[idx2_v.at[pl.ds(lo, half)]], r1_v, sem2)
            c1.wait()
            c2.wait()
            c3 = pltpu.async_copy(r0_v, g0_hbm.at[pl.ds(base + lo, half)], sem1)
            c4 = pltpu.async_copy(r1_v, g1_hbm.at[pl.ds(base + lo, half)], sem2)
            c3.wait()
            c4.wait()

    return k(eo, pos1, pos2)


BT = 512


def _shared_kernel(x_ref, sg_ref, su_ref, sd_ref, seg_ref, g0_ref, g1_ref,
                   w1_ref, w2_ref, out_ref):
    xb = x_ref[...]
    xb_bf = xb.astype(jnp.bfloat16)
    sg = jax.lax.dot_general(
        xb_bf, sg_ref[...], (((1,), (0,)), ((), ())),
        preferred_element_type=jnp.float32)
    su = jax.lax.dot_general(
        xb_bf, su_ref[...], (((1,), (0,)), ((), ())),
        preferred_element_type=jnp.float32)
    sh = jax.nn.silu(sg) * su
    so = jax.lax.dot_general(
        sh.astype(jnp.bfloat16), sd_ref[...], (((1,), (0,)), ((), ())),
        preferred_element_type=jnp.float32)
    glogit = jax.lax.dot_general(
        xb, seg_ref[...], (((1,), (1,)), ((), ())),
        preferred_element_type=jnp.float32)
    out_ref[...] = (jax.nn.sigmoid(glogit) * so
                    + w1_ref[...] * g0_ref[...] + w2_ref[...] * g1_ref[...])


def _shared_call(x, sg_bf, su_bf, sd_bf, seg, g0, g1, w1, w2):
    return pl.pallas_call(
        _shared_kernel,
        grid=(T // BT,),
        in_specs=[
            pl.BlockSpec((BT, D), lambda i: (i, 0)),
            pl.BlockSpec((D, S), lambda i: (0, 0)),
            pl.BlockSpec((D, S), lambda i: (0, 0)),
            pl.BlockSpec((S, D), lambda i: (0, 0)),
            pl.BlockSpec((1, D), lambda i: (0, 0)),
            pl.BlockSpec((BT, D), lambda i: (i, 0)),
            pl.BlockSpec((BT, D), lambda i: (i, 0)),
            pl.BlockSpec((BT, 1), lambda i: (i, 0)),
            pl.BlockSpec((BT, 1), lambda i: (i, 0)),
        ],
        out_specs=pl.BlockSpec((BT, D), lambda i: (i, 0)),
        out_shape=jax.ShapeDtypeStruct((T, D), jnp.float32),
    )(x, sg_bf, su_bf, sd_bf, seg, g0, g1, w1, w2)


def kernel(x, router_weight, expert_gate_up, expert_down, shared_gate,
           shared_up, shared_down, shared_expert_gate):
    egu_bf = expert_gate_up.astype(jnp.bfloat16)
    ed_bf = expert_down.astype(jnp.bfloat16)
    sg_bf = shared_gate.astype(jnp.bfloat16)
    su_bf = shared_up.astype(jnp.bfloat16)
    sd_bf = shared_down.astype(jnp.bfloat16)
    seg = shared_expert_gate.reshape(1, D)

    pos1, pos2, w1, w2, gid = _router_call(x, router_weight)
    pos1_f = pos1.reshape(T)
    pos2_f = pos2.reshape(T)
    gid_f = gid.reshape(MT)

    xs = _sc_dispatch(x, pos1_f, pos2_f)
    eo = _expert_call(gid_f, xs, egu_bf, ed_bf)
    g0, g1 = _sc_combine(eo, pos1_f, pos2_f)
    return _shared_call(x, sg_bf, su_bf, sd_bf, seg, g0, g1, w1, w2)


# speedup vs baseline: 1.2233x; 1.1286x over previous
import functools

import jax
import jax.numpy as jnp
from jax import lax
from jax.experimental import pallas as pl
from jax.experimental.pallas import tpu as pltpu
from jax.experimental.pallas import tpu_sc as plsc

E = 8
TOP_K = 2
D = 1024
I = 512
S = 2048
T = 2048

BM = 256
MT = 24
M_PAD = MT * BM
CB = 256
NC, NS = 2, 16
NW = NC * NS
ROWS_W = T // NW


def _router_kernel(x_ref, rw_ref, pos1_ref, pos2_ref, w1_ref, w2_ref, gid_ref):
    xb = x_ref[...]
    logits = jax.lax.dot_general(
        xb.astype(jnp.bfloat16), rw_ref[...].astype(jnp.bfloat16),
        (((1,), (1,)), ((), ())),
        preferred_element_type=jnp.float32)
    m = jnp.max(logits, axis=-1, keepdims=True)
    ex = jnp.exp(logits - m)
    probs = ex / jnp.sum(ex, axis=-1, keepdims=True)

    eids = jax.lax.broadcasted_iota(jnp.int32, (T, E), 1)
    i1 = jnp.argmax(probs, axis=-1)[:, None]
    oh1 = eids == i1
    v1 = jnp.max(probs, axis=-1, keepdims=True)
    masked = jnp.where(oh1, -jnp.inf, probs)
    i2 = jnp.argmax(masked, axis=-1)[:, None]
    oh2 = eids == i2
    v2 = jnp.max(masked, axis=-1, keepdims=True)
    denom = v1 + v2
    w1_ref[...] = v1 / denom
    w2_ref[...] = v2 / denom

    a = (oh1 | oh2).astype(jnp.float32)
    row = jax.lax.broadcasted_iota(jnp.int32, (CB, CB), 0)
    col = jax.lax.broadcasted_iota(jnp.int32, (CB, CB), 1)
    tri = (row >= col).astype(jnp.bfloat16)
    running = jnp.zeros((1, E), dtype=jnp.float32)
    cex_blocks = []
    for b in range(T // CB):
        ab = a[b * CB:(b + 1) * CB]
        cin = jax.lax.dot_general(
            tri, ab.astype(jnp.bfloat16), (((1,), (0,)), ((), ())),
            preferred_element_type=jnp.float32)
        cex_blocks.append(cin - ab + running)
        running = running + jnp.sum(ab, axis=0, keepdims=True)
    cex = jnp.concatenate(cex_blocks, axis=0)
    counts = running

    nt = jnp.floor((counts + (BM - 1)) * (1.0 / BM))
    erow = jax.lax.broadcasted_iota(jnp.int32, (E, E), 0)
    ecol = jax.lax.broadcasted_iota(jnp.int32, (E, E), 1)
    strict = (erow < ecol).astype(jnp.bfloat16)
    tstart = jax.lax.dot_general(
        nt.astype(jnp.bfloat16), strict, (((1,), (0,)), ((), ())),
        preferred_element_type=jnp.float32)
    padded_off = tstart * float(BM)

    pmat = cex + padded_off
    pos1 = jnp.sum(pmat * oh1.astype(jnp.float32), axis=1, keepdims=True)
    pos2 = jnp.sum(pmat * oh2.astype(jnp.float32), axis=1, keepdims=True)
    pos1_ref[...] = pos1.astype(jnp.int32)
    pos2_ref[...] = pos2.astype(jnp.int32)

    iota_mt = jax.lax.broadcasted_iota(jnp.int32, (1, MT), 1).astype(jnp.float32)
    gidf = jnp.zeros((1, MT), dtype=jnp.float32)
    for e in range(E):
        gidf = gidf + (iota_mt >= tstart[:, e:e + 1]).astype(jnp.float32)
    gid_ref[...] = (gidf - 1.0).astype(jnp.int32)


def _router_call(x, router_weight):
    return pl.pallas_call(
        _router_kernel,
        grid=(1,),
        in_specs=[
            pl.BlockSpec((T, D), lambda i: (0, 0)),
            pl.BlockSpec((E, D), lambda i: (0, 0)),
        ],
        out_specs=[
            pl.BlockSpec((T, 1), lambda i: (0, 0)),
            pl.BlockSpec((T, 1), lambda i: (0, 0)),
            pl.BlockSpec((T, 1), lambda i: (0, 0)),
            pl.BlockSpec((T, 1), lambda i: (0, 0)),
            pl.BlockSpec((1, MT), lambda i: (0, 0)),
        ],
        out_shape=[
            jax.ShapeDtypeStruct((T, 1), jnp.int32),
            jax.ShapeDtypeStruct((T, 1), jnp.int32),
            jax.ShapeDtypeStruct((T, 1), jnp.float32),
            jax.ShapeDtypeStruct((T, 1), jnp.float32),
            jax.ShapeDtypeStruct((1, MT), jnp.int32),
        ],
    )(x, router_weight)


def _sc_dispatch(x, pos1, pos2):
    mesh = plsc.VectorSubcoreMesh(core_axis_name="c", subcore_axis_name="s")

    @functools.partial(
        pl.kernel, mesh=mesh,
        out_type=jax.ShapeDtypeStruct((M_PAD, D), jnp.float32),
        scratch_types=[
            pltpu.VMEM((ROWS_W,), jnp.int32),
            pltpu.VMEM((ROWS_W,), jnp.int32),
            pltpu.VMEM((ROWS_W, D), jnp.float32),
            pltpu.SemaphoreType.DMA,
            pltpu.SemaphoreType.DMA,
        ],
    )
    def k(x_hbm, i1_hbm, i2_hbm, out_hbm, idx1_v, idx2_v, rows_v, sem1, sem2):
        wid = lax.axis_index("s") * NC + lax.axis_index("c")
        base = wid * ROWS_W
        pltpu.sync_copy(i1_hbm.at[pl.ds(base, ROWS_W)], idx1_v)
        pltpu.sync_copy(i2_hbm.at[pl.ds(base, ROWS_W)], idx2_v)
        pltpu.sync_copy(x_hbm.at[pl.ds(base, ROWS_W)], rows_v)
        c1 = pltpu.async_copy(rows_v, out_hbm.at[idx1_v], sem1)
        c2 = pltpu.async_copy(rows_v, out_hbm.at[idx2_v], sem2)
        c1.wait()
        c2.wait()

    return k(x, pos1, pos2)


def _expert_kernel(gid_ref, xs_ref, egu_ref, ed_ref, eo_ref):
    xs = xs_ref[...]
    gu = jax.lax.dot_general(
        xs, egu_ref[0], (((1,), (1,)), ((), ())),
        preferred_element_type=jnp.float32)
    h = jax.nn.silu(gu[:, :I]) * gu[:, I:]
    eo = jax.lax.dot_general(
        h, ed_ref[0], (((1,), (1,)), ((), ())),
        preferred_element_type=jnp.float32)
    eo_ref[...] = eo


def _expert_call(gid, xs, egu_bf, ed_bf):
    grid_spec = pltpu.PrefetchScalarGridSpec(
        num_scalar_prefetch=1,
        grid=(MT,),
        in_specs=[
            pl.BlockSpec((BM, D), lambda i, g: (i, 0)),
            pl.BlockSpec((1, 2 * I, D), lambda i, g: (g[i], 0, 0)),
            pl.BlockSpec((1, D, I), lambda i, g: (g[i], 0, 0)),
        ],
        out_specs=pl.BlockSpec((BM, D), lambda i, g: (i, 0)),
    )
    return pl.pallas_call(
        _expert_kernel,
        grid_spec=grid_spec,
        out_shape=jax.ShapeDtypeStruct((M_PAD, D), jnp.float32),
    )(gid, xs, egu_bf, ed_bf)


def _sc_combine(eo, pos1, pos2):
    mesh = plsc.VectorSubcoreMesh(core_axis_name="c", subcore_axis_name="s")
    row_t = jax.ShapeDtypeStruct((T, D), jnp.float32)

    @functools.partial(
        pl.kernel, mesh=mesh,
        out_type=(row_t, row_t),
        scratch_types=[
            pltpu.VMEM((ROWS_W,), jnp.int32),
            pltpu.VMEM((ROWS_W,), jnp.int32),
            pltpu.VMEM((ROWS_W // 2, D), jnp.float32),
            pltpu.VMEM((ROWS_W // 2, D), jnp.float32),
            pltpu.SemaphoreType.DMA,
            pltpu.SemaphoreType.DMA,
        ],
    )
    def k(eo_hbm, i1_hbm, i2_hbm, g0_hbm, g1_hbm, idx1_v, idx2_v,
          r0_v, r1_v, sem1, sem2):
        wid = lax.axis_index("s") * NC + lax.axis_index("c")
        base = wid * ROWS_W
        half = ROWS_W // 2
        pltpu.sync_copy(i1_hbm.at[pl.ds(base, ROWS_W)], idx1_v)
        pltpu.sync_copy(i2_hbm.at[pl.ds(base, ROWS_W)], idx2_v)
        for chunk in range(2):
            lo = chunk * half
            c1 = pltpu.async_copy(
                eo_hbm.at[idx1_v.at[pl.ds(lo, half)]], r0_v, sem1)
            c2 = pltpu.async_copy(
                eo_hbm.at[idx2_v.at[pl.ds(lo, half)]], r1_v, sem2)
            c1.wait()
            c2.wait()
            c3 = pltpu.async_copy(r0_v, g0_hbm.at[pl.ds(base + lo, half)], sem1)
            c4 = pltpu.async_copy(r1_v, g1_hbm.at[pl.ds(base + lo, half)], sem2)
            c3.wait()
            c4.wait()

    return k(eo, pos1, pos2)


BT = 512


def _shared_kernel(x_ref, sg_ref, su_ref, sd_ref, seg_ref, g0_ref, g1_ref,
                   w1_ref, w2_ref, out_ref):
    xb = x_ref[...]
    sg = jax.lax.dot_general(
        xb, sg_ref[...], (((1,), (0,)), ((), ())),
        preferred_element_type=jnp.float32)
    su = jax.lax.dot_general(
        xb, su_ref[...], (((1,), (0,)), ((), ())),
        preferred_element_type=jnp.float32)
    sh = jax.nn.silu(sg) * su
    so = jax.lax.dot_general(
        sh, sd_ref[...], (((1,), (0,)), ((), ())),
        preferred_element_type=jnp.float32)
    glogit = jax.lax.dot_general(
        xb, seg_ref[...], (((1,), (1,)), ((), ())),
        preferred_element_type=jnp.float32)
    out_ref[...] = (jax.nn.sigmoid(glogit) * so
                    + w1_ref[...] * g0_ref[...] + w2_ref[...] * g1_ref[...])


def _shared_call(x, sg_bf, su_bf, sd_bf, seg, g0, g1, w1, w2):
    return pl.pallas_call(
        _shared_kernel,
        grid=(T // BT,),
        in_specs=[
            pl.BlockSpec((BT, D), lambda i: (i, 0)),
            pl.BlockSpec((D, S), lambda i: (0, 0)),
            pl.BlockSpec((D, S), lambda i: (0, 0)),
            pl.BlockSpec((S, D), lambda i: (0, 0)),
            pl.BlockSpec((1, D), lambda i: (0, 0)),
            pl.BlockSpec((BT, D), lambda i: (i, 0)),
            pl.BlockSpec((BT, D), lambda i: (i, 0)),
            pl.BlockSpec((BT, 1), lambda i: (i, 0)),
            pl.BlockSpec((BT, 1), lambda i: (i, 0)),
        ],
        out_specs=pl.BlockSpec((BT, D), lambda i: (i, 0)),
        out_shape=jax.ShapeDtypeStruct((T, D), jnp.float32),
    )(x, sg_bf, su_bf, sd_bf, seg, g0, g1, w1, w2)


def kernel(x, router_weight, expert_gate_up, expert_down, shared_gate,
           shared_up, shared_down, shared_expert_gate):
    seg = shared_expert_gate.reshape(1, D)

    pos1, pos2, w1, w2, gid = _router_call(x, router_weight)
    pos1_f = pos1.reshape(T)
    pos2_f = pos2.reshape(T)
    gid_f = gid.reshape(MT)

    xs = _sc_dispatch(x, pos1_f, pos2_f)
    eo = _expert_call(gid_f, xs, expert_gate_up, expert_down)
    g0, g1 = _sc_combine(eo, pos1_f, pos2_f)
    return _shared_call(x, shared_gate, shared_up, shared_down, seg, g0, g1, w1, w2)
